# Initial kernel scaffold; baseline (speedup 1.0000x reference)
import jax
import jax.numpy as jnp
from jax.experimental import pallas as pl


def kernel(x, edge_index, W1, b1, W2, b2):
    raise NotImplementedError("write your pallas kernel here")


# speedup vs baseline: 4.2073x; 4.2073x over previous
import jax
import jax.numpy as jnp
from jax import lax
from jax.experimental import pallas as pl
from jax.experimental.pallas import tpu as pltpu
from jax.experimental.pallas import tpu_sc as plsc

N = 10000
E = 320000
D = 128
K_ITERS = 10
ALPHA = 0.1

NC = 2
NS = 16
NW = NC * NS
CHUNK = 128
NCHUNK = 80
EPT = NCHUNK * CHUNK
RPT = 626
ACC_ROWS = NS * RPT
TRASH = N
ZROWS = 64

ROW_BLK = 1000


def _prop_body(src_hbm, dst_hbm, v_hbm, o_hbm,
               src_v, dst_v, stage, zero_v, acc, gsem, ssem):
    c = lax.axis_index("c")
    s = lax.axis_index("s")
    w = c * NS + s

    pltpu.sync_copy(src_hbm.at[w], src_v)
    pltpu.sync_copy(dst_hbm.at[w], dst_v)

    zf = jnp.zeros((16,), jnp.float32)

    def _zrow(i, carry):
        for jj in range(D // 16):
            zero_v[i, pl.ds(jj * 16, 16)] = zf
        return carry

    lax.fori_loop(0, ZROWS, _zrow, 0)
    base = s * RPT
    for k in range(RPT // ZROWS):
        pltpu.sync_copy(zero_v, acc.at[pl.ds(base + k * ZROWS, ZROWS)])
    rem = RPT % ZROWS
    if rem:
        pltpu.sync_copy(zero_v.at[pl.ds(0, rem)],
                        acc.at[pl.ds(base + (RPT // ZROWS) * ZROWS, rem)])
    plsc.subcore_barrier()

    def _edge_chunk(j, carry):
        pltpu.async_copy(v_hbm.at[src_v.at[j]], stage, gsem).wait()
        pltpu.async_copy(stage, acc.at[dst_v.at[j]], ssem, add=True).wait()
        return carry

    lax.fori_loop(0, NCHUNK, _edge_chunk, 0)
    plsc.subcore_barrier()

    pltpu.sync_copy(acc.at[pl.ds(base, RPT)], o_hbm.at[w])


_sc_prop = pl.kernel(
    _prop_body,
    out_type=jax.ShapeDtypeStruct((NW, RPT, D), jnp.float32),
    mesh=plsc.VectorSubcoreMesh(core_axis_name="c", subcore_axis_name="s"),
    scratch_types=[
        pltpu.VMEM((NCHUNK, CHUNK), jnp.int32),
        pltpu.VMEM((NCHUNK, CHUNK), jnp.int32),
        pltpu.VMEM((CHUNK, D), jnp.float32),
        pltpu.VMEM((ZROWS, D), jnp.float32),
        pltpu.VMEM_SHARED((ACC_ROWS, D), jnp.float32),
        pltpu.SemaphoreType.DMA,
        pltpu.SemaphoreType.DMA,
    ],
)


def _rspec(ncols=D):
    return pl.BlockSpec((ROW_BLK, ncols), lambda i: (i, 0))


def _dis_body(deg_ref, dis_ref, dis2_ref, dinv_ref):
    d = deg_ref[...] + 1.0
    r = lax.rsqrt(d)
    dis_ref[...] = r
    dis2_ref[...] = 1.0 / d
    dinv_ref[...] = jnp.sqrt(d)


def _dis_call(deg):
    return pl.pallas_call(
        _dis_body,
        grid=(N // ROW_BLK,),
        in_specs=[_rspec(1)],
        out_specs=[_rspec(1)] * 3,
        out_shape=[jax.ShapeDtypeStruct((N, 1), jnp.float32)] * 3,
    )(deg)


def _mm1_body(x_ref, w_ref, dis_ref, o_ref):
    t = jnp.dot(x_ref[...], w_ref[...], preferred_element_type=jnp.float32)
    o_ref[...] = t * dis_ref[...]


def _mm1_call(x, W1, dis):
    return pl.pallas_call(
        _mm1_body,
        grid=(N // ROW_BLK,),
        in_specs=[_rspec(), pl.BlockSpec((D, D), lambda i: (0, 0)), _rspec(1)],
        out_specs=_rspec(),
        out_shape=jax.ShapeDtypeStruct((N, D), jnp.float32),
    )(x, W1, dis)


def _first_body(a0_ref, a1_ref, t_ref, dis_ref, b1_ref, v_ref, w_ref):
    dis = dis_ref[...]
    h = jnp.maximum(dis * (a0_ref[...] + a1_ref[...] + t_ref[...])
                    + b1_ref[...], 0.0)
    v = dis * h
    v_ref[...] = v
    w_ref[...] = ALPHA * v


def _first_call(a0, a1, t, dis, b1):
    return pl.pallas_call(
        _first_body,
        grid=(N // ROW_BLK,),
        in_specs=[_rspec(), _rspec(), _rspec(), _rspec(1),
                  pl.BlockSpec((1, D), lambda i: (0, 0))],
        out_specs=[_rspec()] * 2,
        out_shape=[jax.ShapeDtypeStruct((N, D), jnp.float32)] * 2,
    )(a0, a1, t, dis, b1)


def _mix_body(a0_ref, a1_ref, v_ref, w_ref, dis2_ref, o_ref):
    f = (1.0 - ALPHA) * dis2_ref[...]
    o_ref[...] = f * (a0_ref[...] + a1_ref[...] + v_ref[...]) + w_ref[...]


def _mix_call(a0, a1, v, w, dis2):
    return pl.pallas_call(
        _mix_body,
        grid=(N // ROW_BLK,),
        in_specs=[_rspec(), _rspec(), _rspec(), _rspec(), _rspec(1)],
        out_specs=_rspec(),
        out_shape=jax.ShapeDtypeStruct((N, D), jnp.float32),
    )(a0, a1, v, w, dis2)


def _mm2_body(v_ref, w_ref, dinv_ref, dis_ref, o_ref):
    h = dinv_ref[...] * v_ref[...]
    g = jnp.dot(h, w_ref[...], preferred_element_type=jnp.float32)
    o_ref[...] = dis_ref[...] * g


def _mm2_call(v, W2, dinv, dis):
    return pl.pallas_call(
        _mm2_body,
        grid=(N // ROW_BLK,),
        in_specs=[_rspec(), pl.BlockSpec((D, D), lambda i: (0, 0)),
                  _rspec(1), _rspec(1)],
        out_specs=_rspec(),
        out_shape=jax.ShapeDtypeStruct((N, D), jnp.float32),
    )(v, W2, dinv, dis)


def _out_body(a0_ref, a1_ref, g_ref, dis_ref, b2_ref, o_ref):
    o = dis_ref[...] * (a0_ref[...] + a1_ref[...] + g_ref[...])
    o_ref[...] = o + b2_ref[...]


def _out_call(a0, a1, g, dis, b2):
    return pl.pallas_call(
        _out_body,
        grid=(N // ROW_BLK,),
        in_specs=[_rspec(), _rspec(), _rspec(), _rspec(1),
                  pl.BlockSpec((1, D), lambda i: (0, 0))],
        out_specs=_rspec(),
        out_shape=jax.ShapeDtypeStruct((N, D), jnp.float32),
    )(a0, a1, g, dis, b2)


def _halves(o):
    a0 = o[:NS].reshape(ACC_ROWS, D)[:N]
    a1 = o[NS:].reshape(ACC_ROWS, D)[:N]
    return a0, a1


def kernel(x, edge_index, W1, b1, W2, b2):
    pad = EPT * NW - E
    src = jnp.concatenate([edge_index[0], jnp.zeros((pad,), jnp.int32)])
    dst = jnp.concatenate([edge_index[1], jnp.full((pad,), TRASH, jnp.int32)])
    src_g = src.reshape(NW, NCHUNK, CHUNK)
    dst_g = dst.reshape(NW, NCHUNK, CHUNK)
    b1r = b1.reshape(1, D)
    b2r = b2.reshape(1, D)

    ones = jnp.ones((N, D), jnp.float32)
    d0, d1 = _halves(_sc_prop(src_g, dst_g, ones))
    deg = (d0[:, 0:1] + d1[:, 0:1])
    dis, dis2, dinv = _dis_call(deg)

    t = _mm1_call(x, W1, dis)
    a0, a1 = _halves(_sc_prop(src_g, dst_g, t))
    v, w = _first_call(a0, a1, t, dis, b1r)

    for _ in range(K_ITERS):
        a0, a1 = _halves(_sc_prop(src_g, dst_g, v))
        v = _mix_call(a0, a1, v, w, dis2)

    g = _mm2_call(v, W2, dinv, dis)
    a0, a1 = _halves(_sc_prop(src_g, dst_g, g))
    return _out_call(a0, a1, g, dis, b2r)


# speedup vs baseline: 4.4590x; 1.0598x over previous
import jax
import jax.numpy as jnp
from jax import lax
from jax.experimental import pallas as pl
from jax.experimental.pallas import tpu as pltpu
from jax.experimental.pallas import tpu_sc as plsc

N = 10000
E = 320000
D = 128
K_ITERS = 10
ALPHA = 0.1

NC = 2
NS = 16
NW = NC * NS
CHUNK = 128
NCHUNK = 80
EPT = NCHUNK * CHUNK
RPT = 626
ACC_ROWS = NS * RPT
TRASH = N

ROW_BLK = 1000


NBUF = 2
HCH0 = NCHUNK // 2
HCH1 = NCHUNK - HCH0
ZROWS = CHUNK


def _prop_body(src_hbm, dst_hbm, v_hbm, o_hbm,
               src_v, dst_v, stage0, stage1, acc, gsem, ssem):
    stage = (stage0, stage1)
    c = lax.axis_index("c")
    s = lax.axis_index("s")
    w = c * NS + s
    base = s * RPT
    ebase = w * NCHUNK

    zf = jnp.zeros((16,), jnp.float32)

    def _zrow(i, carry):
        for jj in range(D // 16):
            stage0[i, pl.ds(jj * 16, 16)] = zf
        return carry

    lax.fori_loop(0, ZROWS, _zrow, 0)
    for k in range(RPT // ZROWS):
        pltpu.sync_copy(stage0, acc.at[pl.ds(base + k * ZROWS, ZROWS)])
    rem = RPT % ZROWS
    if rem:
        pltpu.sync_copy(stage0.at[pl.ds(0, rem)],
                        acc.at[pl.ds(base + (RPT // ZROWS) * ZROWS, rem)])
    plsc.subcore_barrier()

    def _fire_g(j, b):
        pltpu.async_copy(v_hbm.at[src_v.at[j]], stage[b], gsem)

    def _wait_g(j, b):
        pltpu.make_async_copy(v_hbm.at[src_v.at[j]], stage[b], gsem).wait()

    def _fire_s(j, b):
        pltpu.async_copy(stage[b], acc.at[dst_v.at[j]], ssem, add=True)

    def _wait_s(b):
        pltpu.make_async_copy(stage[b], acc.at[dst_v.at[0]], ssem).wait()

    def _half(off, nch):
        pltpu.sync_copy(src_hbm.at[pl.ds(ebase + off, nch)],
                        src_v.at[pl.ds(0, nch)])
        pltpu.sync_copy(dst_hbm.at[pl.ds(ebase + off, nch)],
                        dst_v.at[pl.ds(0, nch)])
        _fire_g(0, 0)
        for j in range(nch):
            b = j % NBUF
            _wait_g(j, b)
            _fire_s(j, b)
            if j + 1 < nch:
                if j >= 1:
                    _wait_s((j + 1) % NBUF)
                _fire_g(j + 1, (j + 1) % NBUF)
        for b in range(NBUF):
            _wait_s(b)

    _half(0, HCH0)
    _half(HCH0, HCH1)

    plsc.subcore_barrier()
    pltpu.sync_copy(acc.at[pl.ds(base, RPT)], o_hbm.at[w])


_sc_prop = pl.kernel(
    _prop_body,
    out_type=jax.ShapeDtypeStruct((NW, RPT, D), jnp.float32),
    mesh=plsc.VectorSubcoreMesh(core_axis_name="c", subcore_axis_name="s"),
    scratch_types=[
        pltpu.VMEM((HCH0, CHUNK), jnp.int32),
        pltpu.VMEM((HCH0, CHUNK), jnp.int32),
        pltpu.VMEM((CHUNK, D), jnp.float32),
        pltpu.VMEM((CHUNK, D), jnp.float32),
        pltpu.VMEM_SHARED((ACC_ROWS, D), jnp.float32),
        pltpu.SemaphoreType.DMA,
        pltpu.SemaphoreType.DMA,
    ],
)


def _rspec(ncols=D):
    return pl.BlockSpec((ROW_BLK, ncols), lambda i: (i, 0))


def _dis_body(deg_ref, dis_ref, dis2_ref, dinv_ref):
    d = deg_ref[...] + 1.0
    r = lax.rsqrt(d)
    dis_ref[...] = r
    dis2_ref[...] = 1.0 / d
    dinv_ref[...] = jnp.sqrt(d)


def _dis_call(deg):
    return pl.pallas_call(
        _dis_body,
        grid=(N // ROW_BLK,),
        in_specs=[_rspec(1)],
        out_specs=[_rspec(1)] * 3,
        out_shape=[jax.ShapeDtypeStruct((N, 1), jnp.float32)] * 3,
    )(deg)


def _mm1_body(x_ref, w_ref, dis_ref, o_ref):
    t = jnp.dot(x_ref[...], w_ref[...], preferred_element_type=jnp.float32)
    o_ref[...] = t * dis_ref[...]


def _mm1_call(x, W1, dis):
    return pl.pallas_call(
        _mm1_body,
        grid=(N // ROW_BLK,),
        in_specs=[_rspec(), pl.BlockSpec((D, D), lambda i: (0, 0)), _rspec(1)],
        out_specs=_rspec(),
        out_shape=jax.ShapeDtypeStruct((N, D), jnp.float32),
    )(x, W1, dis)


def _first_body(a0_ref, a1_ref, t_ref, dis_ref, b1_ref, v_ref, w_ref):
    dis = dis_ref[...]
    h = jnp.maximum(dis * (a0_ref[...] + a1_ref[...] + t_ref[...])
                    + b1_ref[...], 0.0)
    v = dis * h
    v_ref[...] = v
    w_ref[...] = ALPHA * v


def _first_call(a0, a1, t, dis, b1):
    return pl.pallas_call(
        _first_body,
        grid=(N // ROW_BLK,),
        in_specs=[_rspec(), _rspec(), _rspec(), _rspec(1),
                  pl.BlockSpec((1, D), lambda i: (0, 0))],
        out_specs=[_rspec()] * 2,
        out_shape=[jax.ShapeDtypeStruct((N, D), jnp.float32)] * 2,
    )(a0, a1, t, dis, b1)


def _mix_body(a0_ref, a1_ref, v_ref, w_ref, dis2_ref, o_ref):
    f = (1.0 - ALPHA) * dis2_ref[...]
    o_ref[...] = f * (a0_ref[...] + a1_ref[...] + v_ref[...]) + w_ref[...]


def _mix_call(a0, a1, v, w, dis2):
    return pl.pallas_call(
        _mix_body,
        grid=(N // ROW_BLK,),
        in_specs=[_rspec(), _rspec(), _rspec(), _rspec(), _rspec(1)],
        out_specs=_rspec(),
        out_shape=jax.ShapeDtypeStruct((N, D), jnp.float32),
    )(a0, a1, v, w, dis2)


def _mm2_body(v_ref, w_ref, dinv_ref, dis_ref, o_ref):
    h = dinv_ref[...] * v_ref[...]
    g = jnp.dot(h, w_ref[...], preferred_element_type=jnp.float32)
    o_ref[...] = dis_ref[...] * g


def _mm2_call(v, W2, dinv, dis):
    return pl.pallas_call(
        _mm2_body,
        grid=(N // ROW_BLK,),
        in_specs=[_rspec(), pl.BlockSpec((D, D), lambda i: (0, 0)),
                  _rspec(1), _rspec(1)],
        out_specs=_rspec(),
        out_shape=jax.ShapeDtypeStruct((N, D), jnp.float32),
    )(v, W2, dinv, dis)


def _out_body(a0_ref, a1_ref, g_ref, dis_ref, b2_ref, o_ref):
    o = dis_ref[...] * (a0_ref[...] + a1_ref[...] + g_ref[...])
    o_ref[...] = o + b2_ref[...]


def _out_call(a0, a1, g, dis, b2):
    return pl.pallas_call(
        _out_body,
        grid=(N // ROW_BLK,),
        in_specs=[_rspec(), _rspec(), _rspec(), _rspec(1),
                  pl.BlockSpec((1, D), lambda i: (0, 0))],
        out_specs=_rspec(),
        out_shape=jax.ShapeDtypeStruct((N, D), jnp.float32),
    )(a0, a1, g, dis, b2)


def _halves(o):
    a0 = o[:NS].reshape(ACC_ROWS, D)[:N]
    a1 = o[NS:].reshape(ACC_ROWS, D)[:N]
    return a0, a1


def kernel(x, edge_index, W1, b1, W2, b2):
    pad = EPT * NW - E
    src = jnp.concatenate([edge_index[0], jnp.zeros((pad,), jnp.int32)])
    dst = jnp.concatenate([edge_index[1], jnp.full((pad,), TRASH, jnp.int32)])
    src_g = src.reshape(NW * NCHUNK, CHUNK)
    dst_g = dst.reshape(NW * NCHUNK, CHUNK)
    b1r = b1.reshape(1, D)
    b2r = b2.reshape(1, D)

    ones = jnp.ones((N, D), jnp.float32)
    d0, d1 = _halves(_sc_prop(src_g, dst_g, ones))
    deg = d0[:, 0:1] + d1[:, 0:1]
    dis, dis2, dinv = _dis_call(deg)

    t = _mm1_call(x, W1, dis)
    a0, a1 = _halves(_sc_prop(src_g, dst_g, t))
    v, w = _first_call(a0, a1, t, dis, b1r)

    for _ in range(K_ITERS):
        a0, a1 = _halves(_sc_prop(src_g, dst_g, v))
        v = _mix_call(a0, a1, v, w, dis2)

    g = _mm2_call(v, W2, dinv, dis)
    a0, a1 = _halves(_sc_prop(src_g, dst_g, g))
    return _out_call(a0, a1, g, dis, b2r)
